# baseline (device time: 89882 ns/iter reference)
import jax
import jax.numpy as jnp
from jax import lax
from jax.experimental import pallas as pl
from jax.experimental.pallas import tpu as pltpu

N_DEV = 4
SQ_PER = 256
SQ = 1024
D_MODEL = 1024
HQ_PER = 8
DH = 128
NK = 1024
BLK = 64
SCALE = 0.08838834764831843


def kernel(x, Wq, K_ext, V_ext, Wo):
    def body(
        x_ref, wq_ref, k_hbm, v_hbm, wo_ref, out_ref,
        xg_ref, k_ref, v_ref, rs_send, rs_recv,
        wq_bf, wo_bf, k_bf, v_bf, p_mine_ref,
        ag_send_sems, ag_recv_sems, rs_send_sems, rs_recv_sems, kv_sems,
    ):
        my = lax.axis_index("i")
        left = lax.rem(my + N_DEV - 1, N_DEV)
        right = lax.rem(my + 1, N_DEV)
        h0 = my * HQ_PER

        kv_copies = []
        for h in range(HQ_PER):
            ck = pltpu.make_async_copy(
                k_hbm.at[0, pl.ds(0, NK), h0 + h, :], k_ref.at[h], kv_sems.at[h]
            )
            cv = pltpu.make_async_copy(
                v_hbm.at[0, pl.ds(0, NK), h0 + h, :], v_ref.at[h],
                kv_sems.at[HQ_PER + h],
            )
            ck.start()
            cv.start()
            kv_copies += [ck, cv]

        barrier = pltpu.get_barrier_semaphore()
        for nbr in (left, right):
            pl.semaphore_signal(
                barrier, inc=1, device_id=(nbr,),
                device_id_type=pl.DeviceIdType.MESH,
            )
        pl.semaphore_wait(barrier, 2)

        xg_ref[pl.ds(my * SQ_PER, SQ_PER), :] = x_ref[0].astype(jnp.bfloat16)

        def ag_rdma(chunk, hop):
            return pltpu.make_async_remote_copy(
                src_ref=xg_ref.at[pl.ds(chunk * SQ_PER, SQ_PER)],
                dst_ref=xg_ref.at[pl.ds(chunk * SQ_PER, SQ_PER)],
                send_sem=ag_send_sems.at[hop],
                recv_sem=ag_recv_sems.at[hop],
                device_id=(right,),
                device_id_type=pl.DeviceIdType.MESH,
            )

        ag = [ag_rdma(my, 0)]
        ag[0].start()

        wq_bf[:, :] = wq_ref[:, :].astype(jnp.bfloat16)
        wo_bf[:, :] = wo_ref[:, :].astype(jnp.bfloat16)
        for c in kv_copies:
            c.wait()
        k_bf[:, :, :] = k_ref[:, :, :].astype(jnp.bfloat16)
        v_bf[:, :, :] = v_ref[:, :, :].astype(jnp.bfloat16)

        def compute_partial(j):
            nk = SQ_PER * (j + 1)
            xc = xg_ref[j * SQ_PER:(j + 1) * SQ_PER, :]
            q = jnp.dot(xc, wq_bf[:, :], preferred_element_type=jnp.float32)
            q = q.astype(jnp.bfloat16)
            rows = lax.broadcasted_iota(jnp.int32, (SQ_PER, nk), 0)
            kb = lax.broadcasted_iota(jnp.int32, (SQ_PER, nk), 1) // BLK
            mask = kb <= (j * SQ_PER + rows) // BLK
            p = jnp.zeros((SQ_PER, D_MODEL), jnp.float32)
            for h in range(HQ_PER):
                s = lax.dot_general(
                    q[:, h * DH:(h + 1) * DH], k_bf[h, :nk, :],
                    (((1,), (1,)), ((), ())),
                    preferred_element_type=jnp.float32,
                ) * SCALE
                s = jnp.where(mask, s, -1e9)
                m = jnp.max(s, axis=1, keepdims=True)
                w = jnp.exp(s - m)
                ctx_h = lax.dot_general(
                    w.astype(jnp.bfloat16), v_bf[h, :nk, :],
                    (((1,), (0,)), ((), ())),
                    preferred_element_type=jnp.float32,
                )
                denom_inv = 1.0 / jnp.sum(w, axis=1, keepdims=True)
                ctx_h = (ctx_h * denom_inv).astype(jnp.bfloat16)
                p = p + jnp.dot(
                    ctx_h, wo_bf[h * DH:(h + 1) * DH, :],
                    preferred_element_type=jnp.float32,
                )
            return p

        for j in range(N_DEV):
            @pl.when(my == j)
            def _(j=j):
                p_mine_ref[:, :] = compute_partial(j)

        rs = []
        for s in range(N_DEV - 1):
            c = lax.rem(my + 2 * N_DEV - 1 - s, N_DEV)
            ag[s].wait_recv()
            if s < N_DEV - 2:
                ag.append(ag_rdma(c, s + 1))
                ag[s + 1].start()
            if s > 0:
                rs[s - 1].wait_recv()
            for j in range(N_DEV):
                @pl.when(c == j)
                def _(j=j, s=s):
                    p = compute_partial(j)
                    if s == 0:
                        rs_send[s] = p
                    else:
                        rs_send[s] = p + rs_recv[s - 1]
            rs.append(
                pltpu.make_async_remote_copy(
                    src_ref=rs_send.at[s],
                    dst_ref=rs_recv.at[s],
                    send_sem=rs_send_sems.at[s],
                    recv_sem=rs_recv_sems.at[s],
                    device_id=(right,),
                    device_id_type=pl.DeviceIdType.MESH,
                )
            )
            rs[s].start()

        rs[N_DEV - 2].wait_recv()
        out_ref[0] = p_mine_ref[:, :] + rs_recv[N_DEV - 2]

        for r in ag + rs:
            r.wait_send()

    return pl.pallas_call(
        body,
        out_shape=jax.ShapeDtypeStruct((1, SQ_PER, D_MODEL), jnp.float32),
        in_specs=[
            pl.BlockSpec(memory_space=pltpu.VMEM),
            pl.BlockSpec(memory_space=pltpu.VMEM),
            pl.BlockSpec(memory_space=pltpu.MemorySpace.HBM),
            pl.BlockSpec(memory_space=pltpu.MemorySpace.HBM),
            pl.BlockSpec(memory_space=pltpu.VMEM),
        ],
        out_specs=pl.BlockSpec(memory_space=pltpu.VMEM),
        scratch_shapes=[
            pltpu.VMEM((SQ, D_MODEL), jnp.bfloat16),
            pltpu.VMEM((HQ_PER, NK, DH), jnp.float32),
            pltpu.VMEM((HQ_PER, NK, DH), jnp.float32),
            pltpu.VMEM((N_DEV - 1, SQ_PER, D_MODEL), jnp.float32),
            pltpu.VMEM((N_DEV - 1, SQ_PER, D_MODEL), jnp.float32),
            pltpu.VMEM((D_MODEL, HQ_PER * DH), jnp.bfloat16),
            pltpu.VMEM((HQ_PER * DH, D_MODEL), jnp.bfloat16),
            pltpu.VMEM((HQ_PER, NK, DH), jnp.bfloat16),
            pltpu.VMEM((HQ_PER, NK, DH), jnp.bfloat16),
            pltpu.VMEM((SQ_PER, D_MODEL), jnp.float32),
            pltpu.SemaphoreType.DMA((N_DEV - 1,)),
            pltpu.SemaphoreType.DMA((N_DEV - 1,)),
            pltpu.SemaphoreType.DMA((N_DEV - 1,)),
            pltpu.SemaphoreType.DMA((N_DEV - 1,)),
            pltpu.SemaphoreType.DMA((2 * HQ_PER,)),
        ],
        compiler_params=pltpu.CompilerParams(collective_id=0),
    )(x, Wq, K_ext, V_ext, Wo)


# device time: 66867 ns/iter; 1.3442x vs baseline; 1.3442x over previous
import jax
import jax.numpy as jnp
from jax import lax
from jax.experimental import pallas as pl
from jax.experimental.pallas import tpu as pltpu

N_DEV = 4
SQ_PER = 256
SQ = 1024
D_MODEL = 1024
HQ_PER = 8
DH = 128
NK = 1024
BLK = 64
SCALE = 0.08838834764831843


def kernel(x, Wq, K_ext, V_ext, Wo):
    def body(
        x_ref, wq_ref, k_hbm, v_hbm, wo_ref, out_ref,
        xg_ref, k_ref, v_ref, rs_send, rs_recv,
        wq_bf, wo_bf, k_bf, v_bf,
        ag_send_sems, ag_recv_sems, rs_send_sems, rs_recv_sems, kv_sems,
    ):
        my = lax.axis_index("i")
        left = lax.rem(my + N_DEV - 1, N_DEV)
        right = lax.rem(my + 1, N_DEV)
        h0 = my * HQ_PER

        kv_copies = []
        for h in range(HQ_PER):
            ck = pltpu.make_async_copy(
                k_hbm.at[0, pl.ds(0, NK), h0 + h, :], k_ref.at[h], kv_sems.at[h]
            )
            cv = pltpu.make_async_copy(
                v_hbm.at[0, pl.ds(0, NK), h0 + h, :], v_ref.at[h],
                kv_sems.at[HQ_PER + h],
            )
            ck.start()
            cv.start()
            kv_copies += [ck, cv]

        barrier = pltpu.get_barrier_semaphore()
        for nbr in (left, right):
            pl.semaphore_signal(
                barrier, inc=1, device_id=(nbr,),
                device_id_type=pl.DeviceIdType.MESH,
            )
        pl.semaphore_wait(barrier, 2)

        xg_ref[pl.ds(my * SQ_PER, SQ_PER), :] = x_ref[0].astype(jnp.bfloat16)

        def ag_rdma(chunk, hop):
            return pltpu.make_async_remote_copy(
                src_ref=xg_ref.at[pl.ds(chunk * SQ_PER, SQ_PER)],
                dst_ref=xg_ref.at[pl.ds(chunk * SQ_PER, SQ_PER)],
                send_sem=ag_send_sems.at[hop],
                recv_sem=ag_recv_sems.at[hop],
                device_id=(right,),
                device_id_type=pl.DeviceIdType.MESH,
            )

        ag = [ag_rdma(my, 0)]
        ag[0].start()

        wq_bf[:, :] = (wq_ref[:, :] * SCALE).astype(jnp.bfloat16)
        wo_bf[:, :] = wo_ref[:, :].astype(jnp.bfloat16)
        for c in kv_copies:
            c.wait()
        k_bf[:, :, :] = k_ref[:, :, :].astype(jnp.bfloat16)
        v_bf[:, :, :] = v_ref[:, :, :].astype(jnp.bfloat16)

        kb = lax.broadcasted_iota(jnp.int32, (SQ_PER, NK), 1) // BLK
        rows = lax.broadcasted_iota(jnp.int32, (SQ_PER, NK), 0)

        def compute_partial(chunk):
            xc = xg_ref[pl.ds(chunk * SQ_PER, SQ_PER), :]
            q = jnp.dot(xc, wq_bf[:, :], preferred_element_type=jnp.float32)
            q = q.astype(jnp.bfloat16)
            qb = (chunk * SQ_PER + rows) // BLK
            mask = kb <= qb
            p = jnp.zeros((SQ_PER, D_MODEL), jnp.float32)
            for h in range(HQ_PER):
                s = lax.dot_general(
                    q[:, h * DH:(h + 1) * DH], k_bf[h],
                    (((1,), (1,)), ((), ())),
                    preferred_element_type=jnp.float32,
                )
                w = jnp.exp(jnp.where(mask, s, -1e9))
                ctx_h = lax.dot_general(
                    w.astype(jnp.bfloat16), v_bf[h],
                    (((1,), (0,)), ((), ())),
                    preferred_element_type=jnp.float32,
                )
                denom_inv = 1.0 / jnp.sum(w, axis=1, keepdims=True)
                ctx_h = (ctx_h * denom_inv).astype(jnp.bfloat16)
                p = p + jnp.dot(
                    ctx_h, wo_bf[pl.ds(h * DH, DH), :],
                    preferred_element_type=jnp.float32,
                )
            return p

        p_mine = compute_partial(my)

        rs = []
        for s in range(N_DEV - 1):
            c = lax.rem(my + 2 * N_DEV - 1 - s, N_DEV)
            ag[s].wait_recv()
            if s < N_DEV - 2:
                ag.append(ag_rdma(c, s + 1))
                ag[s + 1].start()
            p = compute_partial(c)
            if s == 0:
                rs_send[s] = p
            else:
                rs[s - 1].wait_recv()
                rs_send[s] = p + rs_recv[s - 1]
            rs.append(
                pltpu.make_async_remote_copy(
                    src_ref=rs_send.at[s],
                    dst_ref=rs_recv.at[s],
                    send_sem=rs_send_sems.at[s],
                    recv_sem=rs_recv_sems.at[s],
                    device_id=(right,),
                    device_id_type=pl.DeviceIdType.MESH,
                )
            )
            rs[s].start()

        rs[N_DEV - 2].wait_recv()
        out_ref[0] = p_mine + rs_recv[N_DEV - 2]

        for r in ag + rs:
            r.wait_send()

    return pl.pallas_call(
        body,
        out_shape=jax.ShapeDtypeStruct((1, SQ_PER, D_MODEL), jnp.float32),
        in_specs=[
            pl.BlockSpec(memory_space=pltpu.VMEM),
            pl.BlockSpec(memory_space=pltpu.VMEM),
            pl.BlockSpec(memory_space=pltpu.MemorySpace.HBM),
            pl.BlockSpec(memory_space=pltpu.MemorySpace.HBM),
            pl.BlockSpec(memory_space=pltpu.VMEM),
        ],
        out_specs=pl.BlockSpec(memory_space=pltpu.VMEM),
        scratch_shapes=[
            pltpu.VMEM((SQ, D_MODEL), jnp.bfloat16),
            pltpu.VMEM((HQ_PER, NK, DH), jnp.float32),
            pltpu.VMEM((HQ_PER, NK, DH), jnp.float32),
            pltpu.VMEM((N_DEV - 1, SQ_PER, D_MODEL), jnp.float32),
            pltpu.VMEM((N_DEV - 1, SQ_PER, D_MODEL), jnp.float32),
            pltpu.VMEM((D_MODEL, HQ_PER * DH), jnp.bfloat16),
            pltpu.VMEM((HQ_PER * DH, D_MODEL), jnp.bfloat16),
            pltpu.VMEM((HQ_PER, NK, DH), jnp.bfloat16),
            pltpu.VMEM((HQ_PER, NK, DH), jnp.bfloat16),
            pltpu.SemaphoreType.DMA((N_DEV - 1,)),
            pltpu.SemaphoreType.DMA((N_DEV - 1,)),
            pltpu.SemaphoreType.DMA((N_DEV - 1,)),
            pltpu.SemaphoreType.DMA((N_DEV - 1,)),
            pltpu.SemaphoreType.DMA((2 * HQ_PER,)),
        ],
        compiler_params=pltpu.CompilerParams(collective_id=0),
    )(x, Wq, K_ext, V_ext, Wo)
